# baseline (speedup 1.0000x reference)
import functools

import jax
import jax.numpy as jnp
from jax import lax
from jax.experimental import pallas as pl
from jax.experimental.pallas import tpu as pltpu
from jax.experimental.pallas import tpu_sc as plsc

N_FIELDS = 26
VOCAB = 100000
DIM = 64
BATCH = 4096
TOT = BATCH * N_FIELDS
NC, NS = 2, 16
NW = NC * NS
PER_W = TOT // NW
CHUNK = 128
NCH = PER_W // CHUNK
LANES = 16


def _embed_body(xflat, offs, table, out, idx_v, offs_v, rows_v, gsem):
    wid = lax.axis_index("s") * NC + lax.axis_index("c")
    pltpu.sync_copy(xflat.at[wid], idx_v)
    pltpu.sync_copy(offs, offs_v)

    @pl.loop(0, NCH)
    def _chunk(j):
        for t in range(CHUNK // LANES):
            sl = pl.ds(t * LANES, LANES)
            idx_v[j, sl] = idx_v[j, sl] + offs_v[j, sl]
        pltpu.async_copy(table.at[idx_v.at[j]], rows_v, gsem).wait()
        base = wid * PER_W + j * CHUNK
        pltpu.sync_copy(rows_v, out.at[pl.ds(base, CHUNK)])


@functools.partial(
    pl.kernel,
    out_type=jax.ShapeDtypeStruct((TOT, DIM), jnp.float32),
    mesh=plsc.VectorSubcoreMesh(core_axis_name="c", subcore_axis_name="s"),
    compiler_params=pltpu.CompilerParams(use_tc_tiling_on_sc=False),
    scratch_types=[
        pltpu.VMEM((NCH, CHUNK), jnp.int32),
        pltpu.VMEM((NCH, CHUNK), jnp.int32),
        pltpu.VMEM((CHUNK, DIM), jnp.float32),
        pltpu.SemaphoreType.DMA,
    ],
)
def _embed(xflat, offs, table, out, idx_v, offs_v, rows_v, gsem):
    _embed_body(xflat, offs, table, out, idx_v, offs_v, rows_v, gsem)


def kernel(x, tables):
    xflat = x.reshape(NW, NCH, CHUNK)
    offs = ((jnp.arange(PER_W, dtype=jnp.int32) % N_FIELDS) * VOCAB).reshape(
        NCH, CHUNK)
    table = tables.reshape(N_FIELDS * VOCAB, DIM)
    out = _embed(xflat, offs, table)
    return out.reshape(BATCH, N_FIELDS, DIM)
